# baseline (speedup 1.0000x reference)
import functools

import jax
import jax.numpy as jnp
from jax import lax
from jax.experimental import pallas as pl
from jax.experimental.pallas import tpu as pltpu
from jax.experimental.pallas import tpu_sc as plsc

_NSEG = 16
_SEG = 1024
_D = 512
_L = 16
_QCOL = 128
_CHUNK = 256

_SCSEG = 4
_TCSEG = _NSEG - _SCSEG
_NITEM = _SCSEG * (_D // _QCOL)

_TBLK = 512
_NTB = _SEG // _TBLK


def _sc_body(x_hbm, mean_hbm, in0, in1, ob, sem_in, sem_out):
    sub = lax.axis_index("s")

    @pl.when(sub < 8)
    def _():
        item = lax.axis_index("c") * 8 + sub
        seg = item // 4
        col0 = (item % 4) * _QCOL
        in_bufs = (in0, in1)
        inv = jnp.full((_L,), 1.0 / _SEG, dtype=jnp.float32)
        n_chunks = _SEG // _CHUNK

        def in_copy(chunk, buf):
            return pltpu.make_async_copy(
                x_hbm.at[pl.ds(seg * _SEG + chunk * _CHUNK, _CHUNK),
                         pl.ds(col0, _QCOL)],
                buf, sem_in)

        in_copy(0, in_bufs[0]).start()

        accs = tuple(jnp.zeros((_L,), jnp.float32) for _ in range(8))
        for chunk in range(n_chunks):
            buf = in_bufs[chunk % 2]
            in_copy(chunk, buf).wait()
            if chunk + 1 < n_chunks:
                in_copy(chunk + 1, in_bufs[(chunk + 1) % 2]).start()

            def red_step(t, a):
                r0 = t * 8
                for r in range(8):
                    a = tuple(
                        a[g] + buf[r0 + r, pl.ds(g * _L, _L)]
                        for g in range(8)
                    )
                return a

            accs = lax.fori_loop(0, _CHUNK // 8, red_step, accs)

        means = tuple(a * inv for a in accs)

        for r in range(8):
            for g in range(8):
                ob[r, pl.ds(g * _L, _L)] = means[g]
        pltpu.make_async_copy(
            ob,
            mean_hbm.at[pl.ds(seg * 8, 8), pl.ds(col0, _QCOL)],
            sem_out).start()
        pltpu.make_async_copy(
            ob, mean_hbm.at[pl.ds(0, 8), pl.ds(0, _QCOL)], sem_out
        ).wait()


def _write_seg(v):
    return jnp.where(
        v == 0, 0,
        jnp.where(v <= _TCSEG, v + _SCSEG - 1, v - _TCSEG))


def _tc_fused_body(msc_ref, x_ref, o_ref, acc_ref):
    v = pl.program_id(0)
    c = pl.program_id(1)

    @pl.when((v == 0) & (c == 0))
    def _():
        acc_ref[...] = jnp.zeros_like(acc_ref)

    @pl.when(v < _TCSEG)
    def _():
        seg = _SCSEG + v
        scale = jnp.where(c == _NTB - 1, 1.0 / _SEG, 1.0)
        acc_ref[pl.ds(seg, 1), :] = scale * (
            acc_ref[pl.ds(seg, 1), :]
            + jnp.sum(x_ref[...], axis=0, keepdims=True))

    w = _write_seg(v)

    @pl.when(w < _SCSEG)
    def _():
        o_ref[...] = jnp.broadcast_to(msc_ref[pl.ds(w * 8, 1), :], o_ref.shape)

    @pl.when(w >= _SCSEG)
    def _():
        o_ref[...] = jnp.broadcast_to(acc_ref[pl.ds(w, 1), :], o_ref.shape)


def kernel(embedded_site_features):
    x = embedded_site_features

    mesh = plsc.VectorSubcoreMesh(core_axis_name="c", subcore_axis_name="s")
    sc_run = functools.partial(
        pl.kernel,
        mesh=mesh,
        out_type=jax.ShapeDtypeStruct((_SCSEG * 8, _D), jnp.float32),
        scratch_types=[
            pltpu.VMEM((_CHUNK, _QCOL), jnp.float32),
            pltpu.VMEM((_CHUNK, _QCOL), jnp.float32),
            pltpu.VMEM((8, _QCOL), jnp.float32),
            pltpu.SemaphoreType.DMA,
            pltpu.SemaphoreType.DMA,
        ],
        compiler_params=pltpu.CompilerParams(use_tc_tiling_on_sc=True),
    )(_sc_body)
    sc_means = sc_run(x)

    out = pl.pallas_call(
        _tc_fused_body,
        grid=(_NSEG, _NTB),
        in_specs=[
            pl.BlockSpec((_SCSEG * 8, _D), lambda v, c: (0, 0)),
            pl.BlockSpec(
                (_TBLK, _D),
                lambda v, c: (jnp.where(
                    v < _TCSEG,
                    (_SCSEG + v) * _NTB + c,
                    _NSEG * _NTB - 1), 0)),
        ],
        out_specs=pl.BlockSpec(
            (_TBLK, _D), lambda v, c: (_write_seg(v) * _NTB + c, 0)),
        out_shape=jax.ShapeDtypeStruct((_NSEG * _SEG, _D), jnp.float32),
        scratch_shapes=[pltpu.VMEM((_NSEG, _D), jnp.float32)],
        compiler_params=pltpu.CompilerParams(
            dimension_semantics=("arbitrary", "arbitrary")),
    )(sc_means, x)

    return out


# speedup vs baseline: 1.2454x; 1.2454x over previous
import functools

import jax
import jax.numpy as jnp
from jax import lax
from jax.experimental import pallas as pl
from jax.experimental.pallas import tpu as pltpu
from jax.experimental.pallas import tpu_sc as plsc

_NSEG = 16
_SEG = 1024
_D = 512
_L = 16
_QCOL = 128
_NITEM = _NSEG * (_D // _QCOL)
_CHUNK = 256
_REP = 128


def _body(x_hbm, out_hbm, in0, in1, ob0, ob1, sem_in, sem_out):
    wid = lax.axis_index("c") * 16 + lax.axis_index("s")
    in_bufs = (in0, in1)
    out_bufs = (ob0, ob1)
    inv = jnp.full((_L,), 1.0 / _SEG, dtype=jnp.float32)
    n_chunks = _SEG // _CHUNK
    items = (wid, wid + 32)

    def in_copy(item, chunk, buf):
        seg = item // 4
        col0 = (item % 4) * _QCOL
        return pltpu.make_async_copy(
            x_hbm.at[pl.ds(seg * _SEG + chunk * _CHUNK, _CHUNK),
                     pl.ds(col0, _QCOL)],
            buf, sem_in)

    in_copy(items[0], 0, in_bufs[0]).start()

    for it in range(2):
        item = items[it]
        seg = item // 4
        col0 = (item % 4) * _QCOL
        accs = tuple(jnp.zeros((_L,), jnp.float32) for _ in range(8))
        for chunk in range(n_chunks):
            k = it * n_chunks + chunk
            buf = in_bufs[k % 2]
            in_copy(item, chunk, buf).wait()
            if k + 1 < 2 * n_chunks:
                nk = k + 1
                n_item = items[nk // n_chunks]
                in_copy(n_item, nk % n_chunks, in_bufs[nk % 2]).start()

            def red_step(t, a):
                r0 = t * 8
                for r in range(8):
                    a = tuple(
                        a[g] + buf[r0 + r, pl.ds(g * _L, _L)]
                        for g in range(8)
                    )
                return a

            accs = lax.fori_loop(0, _CHUNK // 8, red_step, accs)

        means = tuple(a * inv for a in accs)

        ob = out_bufs[it]

        def rep_step(i, _):
            for g in range(8):
                ob[i, pl.ds(g * _L, _L)] = means[g]
            return 0

        lax.fori_loop(0, _REP, rep_step, 0)

        for r in range(_SEG // _REP):
            pltpu.make_async_copy(
                ob,
                out_hbm.at[pl.ds(seg * _SEG + r * _REP, _REP),
                           pl.ds(col0, _QCOL)],
                sem_out).start()

    for _ in range(2 * (_SEG // _REP)):
        pltpu.make_async_copy(
            ob0, out_hbm.at[pl.ds(0, _REP), pl.ds(0, _QCOL)], sem_out
        ).wait()


def kernel(embedded_site_features):
    mesh = plsc.VectorSubcoreMesh(core_axis_name="c", subcore_axis_name="s")
    total = _NSEG * _SEG
    run = functools.partial(
        pl.kernel,
        mesh=mesh,
        out_type=jax.ShapeDtypeStruct((total, _D), jnp.float32),
        scratch_types=[
            pltpu.VMEM((_CHUNK, _QCOL), jnp.float32),
            pltpu.VMEM((_CHUNK, _QCOL), jnp.float32),
            pltpu.VMEM((_REP, _QCOL), jnp.float32),
            pltpu.VMEM((_REP, _QCOL), jnp.float32),
            pltpu.SemaphoreType.DMA,
            pltpu.SemaphoreType.DMA,
        ],
        compiler_params=pltpu.CompilerParams(use_tc_tiling_on_sc=True),
    )(_body)
    return run(embedded_site_features)


# speedup vs baseline: 1.2651x; 1.0158x over previous
import functools

import jax
import jax.numpy as jnp
from jax import lax
from jax.experimental import pallas as pl
from jax.experimental.pallas import tpu as pltpu
from jax.experimental.pallas import tpu_sc as plsc

_NSEG = 16
_SEG = 1024
_D = 512
_L = 16
_QCOL = 128
_NITEM = _NSEG * (_D // _QCOL)
_CHUNK = 128
_REP = 128


def _body(x_hbm, out_hbm, in0, in1, in2, ob0, ob1, sem_in, sem_out):
    wid = lax.axis_index("c") * 16 + lax.axis_index("s")
    in_bufs = (in0, in1, in2)
    out_bufs = (ob0, ob1)
    inv = jnp.full((_L,), 1.0 / _SEG, dtype=jnp.float32)
    n_chunks = _SEG // _CHUNK
    items = (wid, wid + 32)

    def in_copy(item, chunk, buf):
        seg = item // 4
        col0 = (item % 4) * _QCOL
        return pltpu.make_async_copy(
            x_hbm.at[pl.ds(seg * _SEG + chunk * _CHUNK, _CHUNK),
                     pl.ds(col0, _QCOL)],
            buf, sem_in)

    in_copy(items[0], 0, in_bufs[0]).start()
    in_copy(items[0], 1, in_bufs[1]).start()

    for it in range(2):
        item = items[it]
        seg = item // 4
        col0 = (item % 4) * _QCOL
        accs = tuple(jnp.zeros((_L,), jnp.float32) for _ in range(8))
        for chunk in range(n_chunks):
            k = it * n_chunks + chunk
            buf = in_bufs[k % 3]
            in_copy(item, chunk, buf).wait()
            if k + 2 < 2 * n_chunks:
                nk = k + 2
                n_item = items[nk // n_chunks]
                in_copy(n_item, nk % n_chunks, in_bufs[nk % 3]).start()

            def red_step(t, a):
                r0 = t * 8
                for r in range(8):
                    a = tuple(
                        a[g] + buf[r0 + r, pl.ds(g * _L, _L)]
                        for g in range(8)
                    )
                return a

            accs = lax.fori_loop(0, _CHUNK // 8, red_step, accs)

        means = tuple(a * inv for a in accs)

        ob = out_bufs[it]

        def rep_step(i, _):
            for g in range(8):
                ob[i, pl.ds(g * _L, _L)] = means[g]
            return 0

        lax.fori_loop(0, _REP, rep_step, 0)

        for r in range(_SEG // _REP):
            pltpu.make_async_copy(
                ob,
                out_hbm.at[pl.ds(seg * _SEG + r * _REP, _REP),
                           pl.ds(col0, _QCOL)],
                sem_out).start()

    for _ in range(2 * (_SEG // _REP)):
        pltpu.make_async_copy(
            ob0, out_hbm.at[pl.ds(0, _REP), pl.ds(0, _QCOL)], sem_out
        ).wait()


def kernel(embedded_site_features):
    mesh = plsc.VectorSubcoreMesh(core_axis_name="c", subcore_axis_name="s")
    total = _NSEG * _SEG
    run = functools.partial(
        pl.kernel,
        mesh=mesh,
        out_type=jax.ShapeDtypeStruct((total, _D), jnp.float32),
        scratch_types=[
            pltpu.VMEM((_CHUNK, _QCOL), jnp.float32),
            pltpu.VMEM((_CHUNK, _QCOL), jnp.float32),
            pltpu.VMEM((_CHUNK, _QCOL), jnp.float32),
            pltpu.VMEM((_REP, _QCOL), jnp.float32),
            pltpu.VMEM((_REP, _QCOL), jnp.float32),
            pltpu.SemaphoreType.DMA,
            pltpu.SemaphoreType.DMA,
        ],
        compiler_params=pltpu.CompilerParams(use_tc_tiling_on_sc=True),
    )(_body)
    return run(embedded_site_features)
